# baseline (speedup 1.0000x reference)
import functools

import jax
import jax.numpy as jnp
from jax import lax
from jax.experimental import pallas as pl
from jax.experimental.pallas import tpu as pltpu, tpu_sc as plsc

NUM_CORES = 2
NUM_SUBCORES = 16
NW = NUM_CORES * NUM_SUBCORES
L = 16
CH = 256


def _sc_gather_wide(sup2d, table2, b_per_w, W):
    mesh = plsc.VectorSubcoreMesh(core_axis_name="c", subcore_axis_name="s")
    n_chunks = b_per_w // CH

    @functools.partial(
        pl.kernel,
        out_type=jax.ShapeDtypeStruct((NW, b_per_w, W), jnp.float32),
        mesh=mesh,
        scratch_types=[
            pltpu.VMEM((b_per_w,), jnp.int32),
            pltpu.VMEM((CH, W), jnp.float32),
            pltpu.SemaphoreType.DMA,
        ],
    )
    def k(table_hbm, idx_hbm, out_hbm, idx_v, rows_v, sem):
        wid = lax.axis_index("s") * NUM_CORES + lax.axis_index("c")
        pltpu.sync_copy(idx_hbm.at[wid], idx_v)

        def chunk(c, _):
            def group(g, _):
                vec = idx_v[pl.ds(c * CH + g * L, L)]
                for l in range(L):
                    r = jnp.squeeze(lax.slice(vec, (l,), (l + 1,)))
                    pltpu.async_copy(table_hbm.at[r], rows_v.at[g * L + l], sem)
                return 0

            lax.fori_loop(0, CH // L, group, 0)
            pltpu.make_async_copy(
                out_hbm.at[wid, pl.ds(c * CH, CH)], rows_v, sem
            ).wait()
            pltpu.sync_copy(rows_v, out_hbm.at[wid, pl.ds(c * CH, CH)])
            return 0

        lax.fori_loop(0, n_chunks, chunk, 0)

    return k(table2, sup2d)


def kernel(labels, train, table):
    B = labels.shape[0]
    V, D = table.shape
    lab = labels.astype(jnp.int32)
    n = (V // 2) * 2
    table2 = jnp.concatenate([table[0:n:2], table[1:n:2]], axis=1)
    b_per_w = B // NW
    sup = (lab >> 1).reshape(NW, b_per_w)
    res = _sc_gather_wide(sup, table2, b_per_w, 2 * D).reshape(B, 2 * D)
    odd = (lab & 1) == 1
    return jnp.where(odd[:, None], res[:, D:], res[:, :D])


# speedup vs baseline: 14.0467x; 14.0467x over previous
import functools

import jax
import jax.numpy as jnp
from jax import lax
from jax.experimental import pallas as pl
from jax.experimental.pallas import tpu as pltpu, tpu_sc as plsc

NUM_CORES = 2
NUM_SUBCORES = 16
NW = NUM_CORES * NUM_SUBCORES
L = 16
CH = 256


def _sc_gather_flat(idx2d, flat, b_per_w, D):
    mesh = plsc.VectorSubcoreMesh(core_axis_name="c", subcore_axis_name="s")
    n_chunks = b_per_w // CH
    B = NW * b_per_w

    @functools.partial(
        pl.kernel,
        out_type=jax.ShapeDtypeStruct((B * D,), jnp.float32),
        mesh=mesh,
        scratch_types=[
            pltpu.VMEM((b_per_w,), jnp.int32),
            pltpu.VMEM((CH * D,), jnp.float32),
            pltpu.SemaphoreType.DMA,
        ],
    )
    def k(flat_hbm, idx_hbm, out_hbm, idx_v, rows_v, sem):
        wid = lax.axis_index("s") * NUM_CORES + lax.axis_index("c")
        pltpu.sync_copy(idx_hbm.at[wid], idx_v)

        def chunk(c, _):
            def group(g, _):
                vec = idx_v[pl.ds(c * CH + g * L, L)]
                for l in range(L):
                    r = jnp.squeeze(lax.slice(vec, (l,), (l + 1,)))
                    pltpu.async_copy(
                        flat_hbm.at[pl.ds(r * D, D)],
                        rows_v.at[pl.ds((g * L + l) * D, D)],
                        sem,
                    )
                return 0

            lax.fori_loop(0, CH // L, group, 0)
            base = (wid * b_per_w + c * CH) * D
            pltpu.make_async_copy(
                out_hbm.at[pl.ds(base, CH * D)], rows_v, sem
            ).wait()
            pltpu.sync_copy(rows_v, out_hbm.at[pl.ds(base, CH * D)])
            return 0

        lax.fori_loop(0, n_chunks, chunk, 0)

    return k(flat, idx2d)


def kernel(labels, train, table):
    B = labels.shape[0]
    V, D = table.shape
    lab = labels.astype(jnp.int32)
    flat = table.reshape(V * D)
    b_per_w = B // NW
    res = _sc_gather_flat(lab.reshape(NW, b_per_w), flat, b_per_w, D)
    return res.reshape(B, D)


# speedup vs baseline: 21.7326x; 1.5472x over previous
import functools

import jax
import jax.numpy as jnp
from jax import lax
from jax.experimental import pallas as pl
from jax.experimental.pallas import tpu as pltpu, tpu_sc as plsc

NUM_CORES = 2
NUM_SUBCORES = 16
NW = NUM_CORES * NUM_SUBCORES
L = 16
CH = 32


def _sc_gather_slab(idx2d, table, b_per_w, D):
    mesh = plsc.VectorSubcoreMesh(core_axis_name="c", subcore_axis_name="s")
    n_chunks = b_per_w // CH

    @functools.partial(
        pl.kernel,
        out_type=jax.ShapeDtypeStruct((NW, b_per_w, D), jnp.float32),
        mesh=mesh,
        scratch_types=[
            pltpu.VMEM((b_per_w,), jnp.int32),
            pltpu.VMEM((CH * 8, D), jnp.float32),
            pltpu.VMEM((CH, D), jnp.float32),
            pltpu.SemaphoreType.DMA,
        ],
    )
    def k(table_hbm, idx_hbm, out_hbm, idx_v, slab_v, rows_v, sem):
        wid = lax.axis_index("s") * NUM_CORES + lax.axis_index("c")
        pltpu.sync_copy(idx_hbm.at[wid], idx_v)

        def chunk(c, _):
            def group(g, _):
                vec = idx_v[pl.ds(c * CH + g * L, L)]
                qvec = jnp.right_shift(vec, 3)
                for l in range(L):
                    q = jnp.squeeze(lax.slice(qvec, (l,), (l + 1,)))
                    pltpu.async_copy(
                        table_hbm.at[pl.ds(q * 8, 8)],
                        slab_v.at[pl.ds((g * L + l) * 8, 8)],
                        sem,
                    )
                return 0

            lax.fori_loop(0, CH // L, group, 0)
            pltpu.make_async_copy(
                table_hbm.at[pl.ds(0, CH * 8)], slab_v, sem
            ).wait()

            for g in range(CH // L):
                vec = idx_v[pl.ds(c * CH + g * L, L)]
                svec = jnp.bitwise_and(vec, 7)
                for l in range(L):
                    s = jnp.squeeze(lax.slice(svec, (l,), (l + 1,)))
                    i = g * L + l
                    for t in range(D // L):
                        rows_v[i, pl.ds(t * L, L)] = slab_v[
                            i * 8 + s, pl.ds(t * L, L)
                        ]

            pltpu.sync_copy(rows_v, out_hbm.at[wid, pl.ds(c * CH, CH)])
            return 0

        lax.fori_loop(0, n_chunks, chunk, 0)

    return k(table, idx2d)


def kernel(labels, train, table):
    B = labels.shape[0]
    V, D = table.shape
    lab = labels.astype(jnp.int32)
    b_per_w = B // NW
    res = _sc_gather_slab(lab.reshape(NW, b_per_w), table, b_per_w, D)
    return res.reshape(B, D)


# speedup vs baseline: 24.0067x; 1.1046x over previous
import functools

import jax
import jax.numpy as jnp
from jax import lax
from jax.experimental import pallas as pl
from jax.experimental.pallas import tpu as pltpu, tpu_sc as plsc

NUM_CORES = 2
NUM_SUBCORES = 16
NW = NUM_CORES * NUM_SUBCORES
L = 16


def _sc_embed(labels2d, table, b_per_w, D):
    mesh = plsc.VectorSubcoreMesh(core_axis_name="c", subcore_axis_name="s")
    n_groups = b_per_w // L

    @functools.partial(
        pl.kernel,
        out_type=jax.ShapeDtypeStruct((NW, b_per_w, D), jnp.float32),
        mesh=mesh,
        scratch_types=[
            pltpu.VMEM((b_per_w,), jnp.int32),
            pltpu.VMEM((b_per_w, D), jnp.float32),
            pltpu.SemaphoreType.DMA,
        ],
    )
    def k(table_hbm, idx_hbm, out_hbm, idx_v, rows_v, sem):
        wid = lax.axis_index("s") * NUM_CORES + lax.axis_index("c")
        pltpu.sync_copy(idx_hbm.at[wid], idx_v)

        def group(g, _):
            vec = idx_v[pl.ds(g * L, L)]
            for l in range(L):
                r = jnp.squeeze(lax.slice(vec, (l,), (l + 1,)))
                pltpu.async_copy(table_hbm.at[r], rows_v.at[g * L + l], sem)
            return 0

        lax.fori_loop(0, n_groups, group, 0)
        pltpu.make_async_copy(out_hbm.at[wid], rows_v, sem).wait()
        pltpu.sync_copy(rows_v, out_hbm.at[wid])

    return k(table, labels2d)


def kernel(labels, train, table):
    B = labels.shape[0]
    V, D = table.shape
    lab = labels.astype(jnp.int32)
    b_per_w = B // NW
    sc_out = _sc_embed(lab.reshape(NW, b_per_w), table, b_per_w, D)
    return sc_out.reshape(B, D)
